# Initial kernel scaffold; baseline (speedup 1.0000x reference)
import functools

import jax
import jax.numpy as jnp
from jax import lax
from jax.experimental import pallas as pl
from jax.experimental.pallas import tpu as pltpu
from jax.experimental.pallas import tpu_sc as plsc

N = 10000
E = 160000
D = 128
A = 4
H = 64

NC = 2
NS = 16
NW = NC * NS
CH = 128
NCHUNK = E // CH
BASE_CHUNKS = NCHUNK // NW
EXTRA = NCHUNK - BASE_CHUNKS * NW
RPT = N // NS

F32 = jnp.float32


def _dot(a, b):
    return jnp.dot(a, b, preferred_element_type=F32)


def _sig(x):
    return 1.0 / (1.0 + jnp.exp(-x))


def _gather_body(nodes_hbm, s_hbm, r_hbm, gs_hbm, gr_hbm,
                 sidx_v, ridx_v, srows_v, rrows_v, sem_s, sem_r):
    c = lax.axis_index("c")
    s = lax.axis_index("s")
    wid = s * NC + c
    nchunks = BASE_CHUNKS + jnp.where(wid < EXTRA, 1, 0)

    def body(i, carry):
        off = (wid + i * NW) * CH
        pltpu.sync_copy(s_hbm.at[pl.ds(off, CH)], sidx_v)
        pltpu.sync_copy(r_hbm.at[pl.ds(off, CH)], ridx_v)
        cp_s = pltpu.async_copy(nodes_hbm.at[sidx_v], srows_v, sem_s)
        cp_r = pltpu.async_copy(nodes_hbm.at[ridx_v], rrows_v, sem_r)
        cp_s.wait()
        cp_r.wait()
        pltpu.sync_copy(srows_v, gs_hbm.at[pl.ds(off, CH)])
        pltpu.sync_copy(rrows_v, gr_hbm.at[pl.ds(off, CH)])
        return carry

    lax.fori_loop(0, nchunks, body, 0)


@jax.jit
def _gather(nodes, senders, receivers):
    return pl.kernel(
        _gather_body,
        mesh=plsc.VectorSubcoreMesh(core_axis_name="c", subcore_axis_name="s"),
        out_type=[
            jax.ShapeDtypeStruct((E, H), F32),
            jax.ShapeDtypeStruct((E, H), F32),
        ],
        scratch_types=[
            pltpu.VMEM((CH,), jnp.int32),
            pltpu.VMEM((CH,), jnp.int32),
            pltpu.VMEM((CH, H), F32),
            pltpu.VMEM((CH, H), F32),
            pltpu.SemaphoreType.DMA,
            pltpu.SemaphoreType.DMA,
        ],
    )(nodes, senders, receivers)


def _scatter_body(msg_hbm, r_hbm, zeros_hbm, out_hbm,
                  idx_v, rows_v, agg_sh, sem):
    c = lax.axis_index("c")
    s = lax.axis_index("s")
    wid = s * NC + c
    nchunks = BASE_CHUNKS + jnp.where(wid < EXTRA, 1, 0)

    pltpu.sync_copy(zeros_hbm.at[pl.ds(s * RPT, RPT)],
                    agg_sh.at[pl.ds(s * RPT, RPT)])
    plsc.subcore_barrier()

    def body(i, carry):
        off = (wid + i * NW) * CH
        pltpu.sync_copy(r_hbm.at[pl.ds(off, CH)], idx_v)
        pltpu.sync_copy(msg_hbm.at[pl.ds(off, CH)], rows_v)
        pltpu.sync_copy(rows_v, agg_sh.at[idx_v], add=True)
        return carry

    lax.fori_loop(0, nchunks, body, 0)
    plsc.subcore_barrier()
    pltpu.sync_copy(agg_sh.at[pl.ds(s * RPT, RPT)],
                    out_hbm.at[c].at[pl.ds(s * RPT, RPT)])


@jax.jit
def _scatter(msg, receivers, zeros):
    return pl.kernel(
        _scatter_body,
        mesh=plsc.VectorSubcoreMesh(core_axis_name="c", subcore_axis_name="s"),
        out_type=jax.ShapeDtypeStruct((NC, N, H), F32),
        scratch_types=[
            pltpu.VMEM((CH,), jnp.int32),
            pltpu.VMEM((CH, H), F32),
            pltpu.VMEM_SHARED((N, H), F32),
            pltpu.SemaphoreType.DMA,
        ],
    )(msg, receivers, zeros)


def _embed_k(x_ref, na_ref, we_ref, ve_ref, out_ref):
    out_ref[...] = _dot(x_ref[...], we_ref[...]) * _dot(na_ref[...], ve_ref[...])


def _embed(x, na, We, Ve):
    return pl.pallas_call(
        _embed_k,
        out_shape=jax.ShapeDtypeStruct((N, H), F32),
    )(x, na, We, Ve)


BM = 5000


def _msg_k(gs_ref, gr_ref, ea_ref, w0a_ref, w0b_ref, v0_ref, w1_ref, v1_ref,
           out_ref):
    h = _dot(gs_ref[...], w0a_ref[...]) + _dot(gr_ref[...], w0b_ref[...])
    h = h * _dot(ea_ref[...], v0_ref[...])
    m = h[:, :H] * _sig(h[:, H:])
    h2 = _dot(m, w1_ref[...]) * _dot(ea_ref[...], v1_ref[...])
    out_ref[...] = h2[:, :H] * _sig(h2[:, H:])


def _messages(gs, gr, ea, W0a, W0b, V0, W1, V1):
    grid = E // BM
    blk = lambda r, c: pl.BlockSpec((r, c), lambda i: (i, 0))
    wblk = lambda r, c: pl.BlockSpec((r, c), lambda i: (0, 0))
    return pl.pallas_call(
        _msg_k,
        grid=(grid,),
        in_specs=[
            blk(BM, H), blk(BM, H), blk(BM, A),
            wblk(H, 2 * H), wblk(H, 2 * H), wblk(A, 2 * H),
            wblk(H, 2 * H), wblk(A, 2 * H),
        ],
        out_specs=blk(BM, H),
        out_shape=jax.ShapeDtypeStruct((E, H), F32),
    )(gs, gr, ea, W0a, W0b, V0, W1, V1)


def _update_k(nodes_ref, agg_ref, na_ref, wa_ref, wb_ref, v0_ref, w1_ref,
              v1_ref, out_ref):
    agg = agg_ref[0] + agg_ref[1]
    h = _dot(nodes_ref[...], wa_ref[...]) + _dot(agg, wb_ref[...])
    h = h * _dot(na_ref[...], v0_ref[...])
    u = h[:, :H] * _sig(h[:, H:])
    upd = _dot(u, w1_ref[...]) * _dot(na_ref[...], v1_ref[...])
    out_ref[...] = nodes_ref[...] + upd


def _update(nodes, agg2, na, Wa, Wb, V0, W1, V1):
    return pl.pallas_call(
        _update_k,
        out_shape=jax.ShapeDtypeStruct((N, H), F32),
    )(nodes, agg2, na, Wa, Wb, V0, W1, V1)


def _dec_k(nodes_ref, na_ref, wp_ref, vp_ref, wpp_ref, vpp_ref, wq_ref,
           wo_ref, out_ref):
    h = _dot(nodes_ref[...], wp_ref[...]) * _dot(na_ref[...], vp_ref[...])
    nd = h[:, :H] * _sig(h[:, H:])
    nd = _dot(nd, wpp_ref[...]) * _dot(na_ref[...], vpp_ref[...])
    g = jnp.sum(nd, axis=0, keepdims=True) * (1.0 / N)
    h2 = _dot(g, wq_ref[...])
    f = h2[:, :H] * _sig(h2[:, H:])
    out_ref[...] = _dot(f, wo_ref[...])


def _decode(nodes, na, Wp, Vp, Wpp, Vpp, Wq, Wo):
    return pl.pallas_call(
        _dec_k,
        out_shape=jax.ShapeDtypeStruct((1, 1), F32),
    )(nodes, na, Wp, Vp, Wpp, Vpp, Wq, Wo)


def kernel(x, node_attr, edge_attr, We, Ve, Wm0, Vm0, Wm1, Vm1, Wu0, Vu0,
           Wu1, Vu1, Wp, Vp, Wpp, Vpp, Wq, Wo, edge_index):
    senders = edge_index[0]
    receivers = edge_index[1]
    zeros = jnp.zeros((N, H), F32)

    nodes = _embed(x, node_attr, We, Ve)
    num_layers = Wm0.shape[0]
    for l in range(num_layers):
        gs, gr = _gather(nodes, senders, receivers)
        msg = _messages(gs, gr, edge_attr,
                        Wm0[l, :H], Wm0[l, H:], Vm0[l], Wm1[l], Vm1[l])
        agg2 = _scatter(msg, receivers, zeros)
        nodes = _update(nodes, agg2, node_attr,
                        Wu0[l, :H], Wu0[l, H:], Vu0[l], Wu1[l], Vu1[l])
    return _decode(nodes, node_attr, Wp, Vp, Wpp, Vpp, Wq, Wo)


# speedup vs baseline: 1.9811x; 1.9811x over previous
import functools

import jax
import jax.numpy as jnp
from jax import lax
from jax.experimental import pallas as pl
from jax.experimental.pallas import tpu as pltpu
from jax.experimental.pallas import tpu_sc as plsc

N = 10000
E = 160000
D = 128
A = 4
H = 64

NC = 2
NS = 16
NW = NC * NS
CH = 128
NCHUNK = E // CH
BASE_CHUNKS = NCHUNK // NW
EXTRA = NCHUNK - BASE_CHUNKS * NW
RPT = N // NS

F32 = jnp.float32


def _dot(a, b):
    return jnp.dot(a, b, preferred_element_type=F32)


def _sig(x):
    return 1.0 / (1.0 + jnp.exp(-x))


def _gather_body(nodes_hbm, s_hbm, r_hbm, gs_hbm, gr_hbm,
                 sidx_v, ridx_v, srows_v, rrows_v, sem_s, sem_r):
    c = lax.axis_index("c")
    s = lax.axis_index("s")
    wid = s * NC + c
    nchunks = BASE_CHUNKS + jnp.where(wid < EXTRA, 1, 0)

    def body(i, carry):
        off = (wid + i * NW) * CH
        pltpu.sync_copy(s_hbm.at[pl.ds(off, CH)], sidx_v)
        pltpu.sync_copy(r_hbm.at[pl.ds(off, CH)], ridx_v)
        cp_s = pltpu.async_copy(nodes_hbm.at[sidx_v], srows_v, sem_s)
        cp_r = pltpu.async_copy(nodes_hbm.at[ridx_v], rrows_v, sem_r)
        cp_s.wait()
        cp_r.wait()
        pltpu.sync_copy(srows_v, gs_hbm.at[pl.ds(off, CH)])
        pltpu.sync_copy(rrows_v, gr_hbm.at[pl.ds(off, CH)])
        return carry

    lax.fori_loop(0, nchunks, body, 0)


@jax.jit
def _gather(nodes, senders, receivers):
    return pl.kernel(
        _gather_body,
        mesh=plsc.VectorSubcoreMesh(core_axis_name="c", subcore_axis_name="s"),
        compiler_params=pltpu.CompilerParams(use_tc_tiling_on_sc=False),
        out_type=[
            jax.ShapeDtypeStruct((E, H), F32),
            jax.ShapeDtypeStruct((E, H), F32),
        ],
        scratch_types=[
            pltpu.VMEM((CH,), jnp.int32),
            pltpu.VMEM((CH,), jnp.int32),
            pltpu.VMEM((CH, H), F32),
            pltpu.VMEM((CH, H), F32),
            pltpu.SemaphoreType.DMA,
            pltpu.SemaphoreType.DMA,
        ],
    )(nodes, senders, receivers)


def _scatter_body(msg_hbm, r_hbm, zeros_hbm, out_hbm,
                  idx_v, rows_v, agg_sh, sem):
    c = lax.axis_index("c")
    s = lax.axis_index("s")
    wid = s * NC + c
    nchunks = BASE_CHUNKS + jnp.where(wid < EXTRA, 1, 0)

    pltpu.sync_copy(zeros_hbm.at[pl.ds(s * RPT, RPT)],
                    agg_sh.at[pl.ds(s * RPT, RPT)])
    plsc.subcore_barrier()

    def body(i, carry):
        off = (wid + i * NW) * CH
        pltpu.sync_copy(r_hbm.at[pl.ds(off, CH)], idx_v)
        pltpu.sync_copy(msg_hbm.at[pl.ds(off, CH)], rows_v)
        pltpu.sync_copy(rows_v, agg_sh.at[idx_v], add=True)
        return carry

    lax.fori_loop(0, nchunks, body, 0)
    plsc.subcore_barrier()
    pltpu.sync_copy(agg_sh.at[pl.ds(s * RPT, RPT)],
                    out_hbm.at[c].at[pl.ds(s * RPT, RPT)])


@jax.jit
def _scatter(msg, receivers, zeros):
    return pl.kernel(
        _scatter_body,
        mesh=plsc.VectorSubcoreMesh(core_axis_name="c", subcore_axis_name="s"),
        compiler_params=pltpu.CompilerParams(use_tc_tiling_on_sc=False),
        out_type=jax.ShapeDtypeStruct((NC, N, H), F32),
        scratch_types=[
            pltpu.VMEM((CH,), jnp.int32),
            pltpu.VMEM((CH, H), F32),
            pltpu.VMEM_SHARED((N, H), F32),
            pltpu.SemaphoreType.DMA,
        ],
    )(msg, receivers, zeros)


def _embed_k(x_ref, na_ref, we_ref, ve_ref, out_ref):
    out_ref[...] = _dot(x_ref[...], we_ref[...]) * _dot(na_ref[...], ve_ref[...])


def _embed(x, na, We, Ve):
    return pl.pallas_call(
        _embed_k,
        out_shape=jax.ShapeDtypeStruct((N, H), F32),
    )(x, na, We, Ve)


BM = 5000


def _msg_k(gs_ref, gr_ref, ea_ref, w0a_ref, w0b_ref, v0_ref, w1_ref, v1_ref,
           out_ref):
    h = _dot(gs_ref[...], w0a_ref[...]) + _dot(gr_ref[...], w0b_ref[...])
    h = h * _dot(ea_ref[...], v0_ref[...])
    m = h[:, :H] * _sig(h[:, H:])
    h2 = _dot(m, w1_ref[...]) * _dot(ea_ref[...], v1_ref[...])
    out_ref[...] = h2[:, :H] * _sig(h2[:, H:])


def _messages(gs, gr, ea, W0a, W0b, V0, W1, V1):
    grid = E // BM
    blk = lambda r, c: pl.BlockSpec((r, c), lambda i: (i, 0))
    wblk = lambda r, c: pl.BlockSpec((r, c), lambda i: (0, 0))
    return pl.pallas_call(
        _msg_k,
        grid=(grid,),
        in_specs=[
            blk(BM, H), blk(BM, H), blk(BM, A),
            wblk(H, 2 * H), wblk(H, 2 * H), wblk(A, 2 * H),
            wblk(H, 2 * H), wblk(A, 2 * H),
        ],
        out_specs=blk(BM, H),
        out_shape=jax.ShapeDtypeStruct((E, H), F32),
    )(gs, gr, ea, W0a, W0b, V0, W1, V1)


def _update_k(nodes_ref, agg_ref, na_ref, wa_ref, wb_ref, v0_ref, w1_ref,
              v1_ref, out_ref):
    agg = agg_ref[0] + agg_ref[1]
    h = _dot(nodes_ref[...], wa_ref[...]) + _dot(agg, wb_ref[...])
    h = h * _dot(na_ref[...], v0_ref[...])
    u = h[:, :H] * _sig(h[:, H:])
    upd = _dot(u, w1_ref[...]) * _dot(na_ref[...], v1_ref[...])
    out_ref[...] = nodes_ref[...] + upd


def _update(nodes, agg2, na, Wa, Wb, V0, W1, V1):
    return pl.pallas_call(
        _update_k,
        out_shape=jax.ShapeDtypeStruct((N, H), F32),
    )(nodes, agg2, na, Wa, Wb, V0, W1, V1)


def _dec_k(nodes_ref, na_ref, wp_ref, vp_ref, wpp_ref, vpp_ref, wq_ref,
           wo_ref, out_ref):
    h = _dot(nodes_ref[...], wp_ref[...]) * _dot(na_ref[...], vp_ref[...])
    nd = h[:, :H] * _sig(h[:, H:])
    nd = _dot(nd, wpp_ref[...]) * _dot(na_ref[...], vpp_ref[...])
    g = jnp.sum(nd, axis=0, keepdims=True) * (1.0 / N)
    h2 = _dot(g, wq_ref[...])
    f = h2[:, :H] * _sig(h2[:, H:])
    out_ref[...] = _dot(f, wo_ref[...])


def _decode(nodes, na, Wp, Vp, Wpp, Vpp, Wq, Wo):
    return pl.pallas_call(
        _dec_k,
        out_shape=jax.ShapeDtypeStruct((1, 1), F32),
    )(nodes, na, Wp, Vp, Wpp, Vpp, Wq, Wo)


def kernel(x, node_attr, edge_attr, We, Ve, Wm0, Vm0, Wm1, Vm1, Wu0, Vu0,
           Wu1, Vu1, Wp, Vp, Wpp, Vpp, Wq, Wo, edge_index):
    senders = edge_index[0]
    receivers = edge_index[1]
    zeros = jnp.zeros((N, H), F32)

    nodes = _embed(x, node_attr, We, Ve)
    num_layers = Wm0.shape[0]
    for l in range(num_layers):
        gs, gr = _gather(nodes, senders, receivers)
        msg = _messages(gs, gr, edge_attr,
                        Wm0[l, :H], Wm0[l, H:], Vm0[l], Wm1[l], Vm1[l])
        agg2 = _scatter(msg, receivers, zeros)
        nodes = _update(nodes, agg2, node_attr,
                        Wu0[l, :H], Wu0[l, H:], Vu0[l], Wu1[l], Vu1[l])
    return _decode(nodes, node_attr, Wp, Vp, Wpp, Vpp, Wq, Wo)
